# Initial kernel scaffold; baseline (speedup 1.0000x reference)
import jax
import jax.numpy as jnp
from jax.experimental import pallas as pl


def kernel(x):
    raise NotImplementedError("write your pallas kernel here")


# speedup vs baseline: 19.3114x; 19.3114x over previous
import functools

import jax
import jax.numpy as jnp
from jax import lax
from jax.experimental import pallas as pl
from jax.experimental.pallas import tpu as pltpu
from jax.experimental.pallas import tpu_sc as plsc

B, C, T = 4, 8, 150050
PS, ST, P = 200, 150, 1000
NW = 32
PC = 200
NCHUNK = P // PC
CIN = ST * PC + (PS - ST)
CINB = CIN + 14
COUT = PS * PC
LANES = 16

_mesh = plsc.VectorSubcoreMesh(core_axis_name="c", subcore_axis_name="s")


@functools.partial(
    pl.kernel,
    mesh=_mesh,
    out_type=[
        jax.ShapeDtypeStruct((NW * P * PS,), jnp.float32),
        jax.ShapeDtypeStruct((NW * P,), jnp.float32),
    ],
    scratch_types=[
        pltpu.VMEM((CINB,), jnp.float32),
        pltpu.VMEM((COUT,), jnp.float32),
        pltpu.VMEM((P,), jnp.float32),
    ],
)
def _to_patches_sc(x_hbm, patches_hbm, masks_hbm, in_buf, out_buf, mask_buf):
    wid = lax.axis_index("s") * 2 + lax.axis_index("c")
    base = wid * T
    r8 = base % 8
    al_base = pl.multiple_of(base - r8, 8)
    out_base = pl.multiple_of(wid * (P * PS), 8)

    ones = jnp.ones((LANES,), jnp.float32)
    for j in range(P // LANES):
        mask_buf[pl.ds(j * LANES, LANES)] = ones
    mask_buf[pl.ds(P - LANES, LANES)] = ones
    pltpu.sync_copy(
        mask_buf, masks_hbm.at[pl.ds(pl.multiple_of(wid * P, 8), P)]
    )

    def densify(d):
        def patch_body(i, carry):
            src = d + i * ST
            dst = i * PS
            for v in range(PS // LANES):
                out_buf[pl.ds(dst + v * LANES, LANES)] = in_buf[
                    pl.ds(src + v * LANES, LANES)
                ]
            out_buf[pl.ds(dst + PS - LANES, LANES)] = in_buf[
                pl.ds(src + PS - LANES, LANES)
            ]
            return carry

        lax.fori_loop(0, PC, patch_body, 0)

    def chunk_body(k, carry):
        start = pl.multiple_of(al_base + k * (ST * PC), 8)
        pltpu.sync_copy(x_hbm.at[pl.ds(start, CINB)], in_buf)
        densify(r8)
        pltpu.sync_copy(
            out_buf,
            patches_hbm.at[pl.ds(pl.multiple_of(out_base + k * COUT, 8), COUT)],
        )
        return carry

    lax.fori_loop(0, NCHUNK - 1, chunk_body, 0)

    last_lo = base + (NCHUNK - 1) * (ST * PC)
    start = pl.multiple_of(base + T - CIN - r8 - 8, 8)
    pltpu.sync_copy(x_hbm.at[pl.ds(start, CINB)], in_buf)
    densify(last_lo - start)
    pltpu.sync_copy(
        out_buf,
        patches_hbm.at[
            pl.ds(pl.multiple_of(out_base + (NCHUNK - 1) * COUT, 8), COUT)
        ],
    )


def kernel(x):
    patches, masks = _to_patches_sc(x.reshape(NW * T))
    return patches.reshape(B, C, P, PS), masks.reshape(B, C, P)


# speedup vs baseline: 29.4299x; 1.5240x over previous
import functools

import jax
import jax.numpy as jnp
from jax import lax
from jax.experimental import pallas as pl
from jax.experimental.pallas import tpu as pltpu
from jax.experimental.pallas import tpu_sc as plsc

B, C, T = 4, 8, 150050
PS, ST, P = 200, 150, 1000
NW = 32
LANES = 16

SUB = 16
NSUB = 8
WIN = SUB * NSUB
CLEN = 2688
NV = CLEN // LANES
CLAMP = 976
OUTC = SUB * PS

TAIL_P0 = 992
TAIL_NP = P - TAIL_P0
TAIL_SRC = 148786
TAIL_D = TAIL_P0 * ST - TAIL_SRC
TAIL_LEN = T - TAIL_SRC
NVT = TAIL_LEN // LANES

OFFS = [v * LANES for v in range(PS // LANES)] + [PS - LANES]

_mesh = plsc.VectorSubcoreMesh(core_axis_name="c", subcore_axis_name="s")


@functools.partial(
    pl.kernel,
    mesh=_mesh,
    out_type=[
        jax.ShapeDtypeStruct((B * C * P * PS,), jnp.float32),
        jax.ShapeDtypeStruct((B, C, P), jnp.float32),
    ],
    scratch_types=[
        pltpu.VMEM((C, CLEN), jnp.float32),
        pltpu.VMEM((C * CLEN,), jnp.float32),
        pltpu.VMEM((C * OUTC,), jnp.float32),
        pltpu.VMEM((C, P), jnp.float32),
        pltpu.VMEM((C, TAIL_LEN), jnp.float32),
        pltpu.SemaphoreType.DMA,
    ],
)
def _to_patches_sc(
    x_hbm, xtail_hbm, patches_hbm, masks_hbm,
    in_buf, lin, outl, ones2d, tailb, sem,
):
    wid = lax.axis_index("s") * 2 + lax.axis_index("c")
    b = wid // NSUB
    j = wid % NSUB

    ones = jnp.ones((LANES,), jnp.float32)

    @pl.when(j == 0)
    def _write_masks():
        def fill(i, carry):
            col = pl.multiple_of(i * LANES, LANES)
            for c in range(C):
                ones2d[c, pl.ds(col, LANES)] = ones
            return carry

        lax.fori_loop(0, P // LANES, fill, 0)
        for c in range(C):
            ones2d[c, pl.ds(P - LANES, LANES)] = ones
        pltpu.sync_copy(ones2d, masks_hbm.at[b])

    def flatten(src2d, nvec):
        def fv(v, carry):
            col = pl.multiple_of(v * LANES, LANES)
            for c in range(C):
                lin[pl.ds(c * CLEN + col, LANES)] = src2d[c, pl.ds(col, LANES)]
            return carry

        lax.fori_loop(0, nvec, fv, 0)

    def densify(d, npatch):
        def fp(i, carry):
            src0 = d + i * ST
            dst0 = i * PS
            for c in range(C):
                cb = c * CLEN
                ob = c * OUTC
                for off in OFFS:
                    outl[pl.ds(ob + dst0 + off, LANES)] = lin[
                        pl.ds(cb + src0 + off, LANES)
                    ]
            return carry

        lax.fori_loop(0, npatch, fp, 0)

    def store_out(p0, npatch):
        n = npatch * PS
        cps = [
            pltpu.async_copy(
                outl.at[pl.ds(c * OUTC, n)],
                patches_hbm.at[
                    pl.ds(pl.multiple_of(((b * C + c) * P + p0) * PS, 8), n)
                ],
                sem,
            )
            for c in range(C)
        ]
        for cp in cps:
            cp.wait()

    def subchunk(s, carry):
        p0 = jnp.minimum(WIN * j + SUB * s, CLAMP)
        src = p0 * ST
        cstart = pl.multiple_of((src // 128) * 128, 128)
        pltpu.sync_copy(x_hbm.at[b, :, pl.ds(cstart, CLEN)], in_buf)
        flatten(in_buf, NV)
        densify(src - cstart, SUB)
        store_out(p0, SUB)
        return carry

    lax.fori_loop(0, NSUB, subchunk, 0)

    @pl.when(j == NSUB - 1)
    def _tail():
        pltpu.sync_copy(xtail_hbm.at[b], tailb)
        flatten(tailb, NVT)
        densify(TAIL_D, TAIL_NP)
        store_out(TAIL_P0, TAIL_NP)


def kernel(x):
    xtail = x[:, :, TAIL_SRC:]
    patches, masks = _to_patches_sc(x, xtail)
    return patches.reshape(B, C, P, PS), masks


# speedup vs baseline: 49.0927x; 1.6681x over previous
import functools

import jax
import jax.numpy as jnp
from jax import lax
from jax.experimental import pallas as pl
from jax.experimental.pallas import tpu as pltpu
from jax.experimental.pallas import tpu_sc as plsc

B, C, T = 4, 8, 150050
PS, ST, P = 200, 150, 1000
NW = 32
LANES = 16

SUB = 16
NSUB = 8
WIN = SUB * NSUB
CLEN = 2688
NV = CLEN // LANES
CLAMP = 976
OUTC = SUB * PS

PSP = 256
TAIL_P0 = 992
TAIL_NP = P - TAIL_P0
TAIL_SRC = 148786
TAIL_D = TAIL_P0 * ST - TAIL_SRC
TAIL_LEN = 1280
NVT = TAIL_LEN // LANES

OFFS = [v * LANES for v in range(13)]

_mesh = plsc.VectorSubcoreMesh(core_axis_name="c", subcore_axis_name="s")


@functools.partial(
    pl.kernel,
    mesh=_mesh,
    out_type=[
        jax.ShapeDtypeStruct((B * C * P, PSP), jnp.float32),
        jax.ShapeDtypeStruct((B, C, P), jnp.float32),
    ],
    scratch_types=[
        pltpu.VMEM((C, CLEN), jnp.float32),
        pltpu.VMEM((C * CLEN,), jnp.float32),
        pltpu.VMEM((C, SUB, PSP), jnp.float32),
        pltpu.VMEM((C, P), jnp.float32),
        pltpu.VMEM((C, TAIL_LEN), jnp.float32),
        pltpu.SemaphoreType.DMA,
    ],
)
def _to_patches_sc(
    x_hbm, xtail_hbm, patches_hbm, masks_hbm,
    in_buf, lin, outb, ones2d, tailb, sem,
):
    wid = lax.axis_index("s") * 2 + lax.axis_index("c")
    b = wid // NSUB
    j = wid % NSUB

    ones = jnp.ones((LANES,), jnp.float32)

    @pl.when(j == 0)
    def _write_masks():
        def fill(i, carry):
            col = pl.multiple_of(i * LANES, LANES)
            for c in range(C):
                ones2d[c, pl.ds(col, LANES)] = ones
            return carry

        lax.fori_loop(0, P // LANES, fill, 0)
        for c in range(C):
            ones2d[c, pl.ds(P - LANES, LANES)] = ones
        pltpu.sync_copy(ones2d, masks_hbm.at[b])

    def flatten(src2d, nvec):
        def fv(v, carry):
            col = pl.multiple_of(v * LANES, LANES)
            for c in range(C):
                lin[pl.ds(c * CLEN + col, LANES)] = src2d[c, pl.ds(col, LANES)]
            return carry

        lax.fori_loop(0, nvec, fv, 0)

    def densify(d, npatch):
        def fc(c, carry):
            cb = c * CLEN + d
            for i in range(npatch):
                src0 = cb + i * ST
                for off in OFFS:
                    outb[c, i, pl.ds(off, LANES)] = lin[pl.ds(src0 + off, LANES)]
            return carry

        lax.fori_loop(0, C, fc, 0)

    def store_out(p0, npatch):
        cps = [
            pltpu.async_copy(
                outb.at[c, pl.ds(0, npatch), :],
                patches_hbm.at[
                    pl.ds(pl.multiple_of((b * C + c) * P + p0, 8), npatch), :
                ],
                sem,
            )
            for c in range(C)
        ]
        for cp in cps:
            cp.wait()

    def subchunk(s, carry):
        p0 = jnp.minimum(WIN * j + SUB * s, CLAMP)
        src = p0 * ST
        cstart = pl.multiple_of((src // 128) * 128, 128)
        pltpu.sync_copy(x_hbm.at[b, :, pl.ds(cstart, CLEN)], in_buf)
        flatten(in_buf, NV)
        densify(src - cstart, SUB)
        store_out(p0, SUB)
        return carry

    lax.fori_loop(0, NSUB, subchunk, 0)

    @pl.when(j == NSUB - 1)
    def _tail():
        pltpu.sync_copy(xtail_hbm.at[b], tailb)
        flatten(tailb, NVT)
        densify(TAIL_D, TAIL_NP)
        store_out(TAIL_P0, TAIL_NP)


def _tc_transpose_body(x_ref, o_ref):
    o_ref[0, 0] = jnp.transpose(x_ref[:, :PS], (1, 0))


_tc_transpose = pl.pallas_call(
    _tc_transpose_body,
    grid=(B, C),
    in_specs=[pl.BlockSpec((P, PSP), lambda b, c: (b * C + c, 0))],
    out_specs=pl.BlockSpec((1, 1, PS, P), lambda b, c: (b, c, 0, 0)),
    out_shape=jax.ShapeDtypeStruct((B, C, PS, P), jnp.float32),
)


def kernel(x):
    xtail = jnp.pad(x[:, :, TAIL_SRC:], ((0, 0), (0, 0), (0, TAIL_LEN - (T - TAIL_SRC))))
    patches2d, masks = _to_patches_sc(x, xtail)
    patches_t = _tc_transpose(patches2d)
    return patches_t.transpose(0, 1, 3, 2), masks


# speedup vs baseline: 56.3375x; 1.1476x over previous
import functools

import jax
import jax.numpy as jnp
from jax import lax
from jax.experimental import pallas as pl
from jax.experimental.pallas import tpu as pltpu
from jax.experimental.pallas import tpu_sc as plsc

B, C, T = 4, 8, 150050
PS, ST, P = 200, 150, 1000
NW = 32
LANES = 16

SUB = 16
NSUB = 8
WIN = SUB * NSUB
CLEN = 2688
NV = CLEN // LANES
CLAMP = 976
OUTC = SUB * PS

PSP = 256
TAIL_P0 = 992
TAIL_NP = P - TAIL_P0
TAIL_SRC = 148786
TAIL_D = TAIL_P0 * ST - TAIL_SRC
TAIL_LEN = 1280
NVT = TAIL_LEN // LANES

OFFS = [v * LANES for v in range(13)]

_mesh = plsc.VectorSubcoreMesh(core_axis_name="c", subcore_axis_name="s")


@functools.partial(
    pl.kernel,
    mesh=_mesh,
    out_type=[
        jax.ShapeDtypeStruct((B * C * P, PSP), jnp.float32),
        jax.ShapeDtypeStruct((B, C, P), jnp.float32),
    ],
    scratch_types=[
        pltpu.VMEM((2, C, CLEN), jnp.float32),
        pltpu.VMEM((C * CLEN,), jnp.float32),
        pltpu.VMEM((C, SUB, PSP), jnp.float32),
        pltpu.VMEM((C, P), jnp.float32),
        pltpu.VMEM((C, TAIL_LEN), jnp.float32),
        pltpu.SemaphoreType.DMA,
        pltpu.SemaphoreType.DMA,
    ],
)
def _to_patches_sc(
    x_hbm, xtail_hbm, patches_hbm, masks_hbm,
    in2, lin, outb, ones2d, tailb, sem_in, sem_out,
):
    wid = lax.axis_index("s") * 2 + lax.axis_index("c")
    b = wid // NSUB
    j = wid % NSUB

    ones = jnp.ones((LANES,), jnp.float32)

    @pl.when(j == 0)
    def _write_masks():
        def fill(i, carry):
            col = pl.multiple_of(i * LANES, LANES)
            for c in range(C):
                ones2d[c, pl.ds(col, LANES)] = ones
            return carry

        lax.fori_loop(0, P // LANES, fill, 0)
        for c in range(C):
            ones2d[c, pl.ds(P - LANES, LANES)] = ones
        pltpu.sync_copy(ones2d, masks_hbm.at[b])

    def flatten(bi, nvec):
        def fv(v, carry):
            col = pl.multiple_of(v * LANES, LANES)
            for c in range(C):
                lin[pl.ds(c * CLEN + col, LANES)] = in2[bi, c, pl.ds(col, LANES)]
            return carry

        lax.fori_loop(0, nvec, fv, 0)

    def flatten_tail(nvec):
        def fv(v, carry):
            col = pl.multiple_of(v * LANES, LANES)
            for c in range(C):
                lin[pl.ds(c * CLEN + col, LANES)] = tailb[c, pl.ds(col, LANES)]
            return carry

        lax.fori_loop(0, nvec, fv, 0)

    def densify(d, npatch):
        def fp(i, carry):
            src0 = d + i * ST
            for c in range(C):
                cb = c * CLEN + src0
                for off in OFFS:
                    outb[c, i, pl.ds(off, LANES)] = lin[pl.ds(cb + off, LANES)]
            return carry

        lax.fori_loop(0, npatch, fp, 0)

    def fire_outs(p0, npatch):
        return [
            pltpu.async_copy(
                outb.at[c, pl.ds(0, npatch), :],
                patches_hbm.at[
                    pl.ds(pl.multiple_of((b * C + c) * P + p0, 8), npatch), :
                ],
                sem_out,
            )
            for c in range(C)
        ]

    def fire_load(s):
        p0 = jnp.minimum(WIN * j + SUB * s, CLAMP)
        src = p0 * ST
        cstart = pl.multiple_of((src // 128) * 128, 128)
        cp = pltpu.async_copy(
            x_hbm.at[b, :, pl.ds(cstart, CLEN)], in2.at[s % 2], sem_in
        )
        return cp, src - cstart, p0

    load = fire_load(0)
    outs = None
    for s in range(NSUB):
        cp, d, p0 = load
        cp.wait()
        if s + 1 < NSUB:
            load = fire_load(s + 1)
        flatten(s % 2, NV)
        if outs is not None:
            for ocp in outs:
                ocp.wait()
        densify(d, SUB)
        outs = fire_outs(p0, SUB)
    for ocp in outs:
        ocp.wait()

    @pl.when(j == NSUB - 1)
    def _tail():
        pltpu.sync_copy(xtail_hbm.at[b], tailb)
        flatten_tail(NVT)
        densify(TAIL_D, TAIL_NP)
        for ocp in fire_outs(TAIL_P0, TAIL_NP):
            ocp.wait()


def _tc_transpose_body(x_ref, o_ref):
    o_ref[0, 0] = jnp.transpose(x_ref[:, :PS], (1, 0))


_tc_transpose = pl.pallas_call(
    _tc_transpose_body,
    grid=(B, C),
    in_specs=[pl.BlockSpec((P, PSP), lambda b, c: (b * C + c, 0))],
    out_specs=pl.BlockSpec((1, 1, PS, P), lambda b, c: (b, c, 0, 0)),
    out_shape=jax.ShapeDtypeStruct((B, C, PS, P), jnp.float32),
)


def kernel(x):
    xtail = jnp.pad(x[:, :, TAIL_SRC:], ((0, 0), (0, 0), (0, TAIL_LEN - (T - TAIL_SRC))))
    patches2d, masks = _to_patches_sc(x, xtail)
    patches_t = _tc_transpose(patches2d)
    return patches_t.transpose(0, 1, 3, 2), masks


# speedup vs baseline: 56.4903x; 1.0027x over previous
import functools

import jax
import jax.numpy as jnp
from jax import lax
from jax.experimental import pallas as pl
from jax.experimental.pallas import tpu as pltpu
from jax.experimental.pallas import tpu_sc as plsc

B, C, T = 4, 8, 150050
PS, ST, P = 200, 150, 1000
NW = 32
LANES = 16

SUB = 16
NSUB = 8
WIN = SUB * NSUB
CLEN = 2688
NV = CLEN // LANES
CLAMP = 976
OUTC = SUB * PS

PSP = 208
TAIL_P0 = 992
TAIL_NP = P - TAIL_P0
TAIL_SRC = 148786
TAIL_D = TAIL_P0 * ST - TAIL_SRC
TAIL_LEN = 1280
NVT = TAIL_LEN // LANES

OFFS = [v * LANES for v in range(13)]

_mesh = plsc.VectorSubcoreMesh(core_axis_name="c", subcore_axis_name="s")


@functools.partial(
    pl.kernel,
    mesh=_mesh,
    out_type=[
        jax.ShapeDtypeStruct((B * C * P, PSP), jnp.float32),
        jax.ShapeDtypeStruct((B, C, P), jnp.float32),
    ],
    scratch_types=[
        pltpu.VMEM((2, C, CLEN), jnp.float32),
        pltpu.VMEM((C * CLEN,), jnp.float32),
        pltpu.VMEM((C, SUB, PSP), jnp.float32),
        pltpu.VMEM((C, P), jnp.float32),
        pltpu.VMEM((C, TAIL_LEN), jnp.float32),
        pltpu.SemaphoreType.DMA,
        pltpu.SemaphoreType.DMA,
    ],
)
def _to_patches_sc(
    x_hbm, xtail_hbm, patches_hbm, masks_hbm,
    in2, lin, outb, ones2d, tailb, sem_in, sem_out,
):
    wid = lax.axis_index("s") * 2 + lax.axis_index("c")
    b = wid // NSUB
    j = wid % NSUB

    ones = jnp.ones((LANES,), jnp.float32)

    @pl.when(j == 0)
    def _write_masks():
        def fill(i, carry):
            col = pl.multiple_of(i * LANES, LANES)
            for c in range(C):
                ones2d[c, pl.ds(col, LANES)] = ones
            return carry

        lax.fori_loop(0, P // LANES, fill, 0)
        for c in range(C):
            ones2d[c, pl.ds(P - LANES, LANES)] = ones
        pltpu.sync_copy(ones2d, masks_hbm.at[b])

    def flatten(bi, nvec):
        def fv(v, carry):
            col = pl.multiple_of(v * LANES, LANES)
            for c in range(C):
                lin[pl.ds(c * CLEN + col, LANES)] = in2[bi, c, pl.ds(col, LANES)]
            return carry

        lax.fori_loop(0, nvec, fv, 0)

    def flatten_tail(nvec):
        def fv(v, carry):
            col = pl.multiple_of(v * LANES, LANES)
            for c in range(C):
                lin[pl.ds(c * CLEN + col, LANES)] = tailb[c, pl.ds(col, LANES)]
            return carry

        lax.fori_loop(0, nvec, fv, 0)

    def densify(d, npatch):
        def fp(i, carry):
            src0 = d + i * ST
            for c in range(C):
                cb = c * CLEN + src0
                for off in OFFS:
                    outb[c, i, pl.ds(off, LANES)] = lin[pl.ds(cb + off, LANES)]
            return carry

        lax.fori_loop(0, npatch, fp, 0)

    def fire_outs(p0, npatch):
        return [
            pltpu.async_copy(
                outb.at[c, pl.ds(0, npatch), :],
                patches_hbm.at[
                    pl.ds(pl.multiple_of((b * C + c) * P + p0, 8), npatch), :
                ],
                sem_out,
            )
            for c in range(C)
        ]

    def fire_load(s):
        p0 = jnp.minimum(WIN * j + SUB * s, CLAMP)
        src = p0 * ST
        cstart = pl.multiple_of((src // 128) * 128, 128)
        cp = pltpu.async_copy(
            x_hbm.at[b, :, pl.ds(cstart, CLEN)], in2.at[s % 2], sem_in
        )
        return cp, src - cstart, p0

    load = fire_load(0)
    outs = None
    for s in range(NSUB):
        cp, d, p0 = load
        cp.wait()
        if s + 1 < NSUB:
            load = fire_load(s + 1)
        flatten(s % 2, NV)
        if outs is not None:
            for ocp in outs:
                ocp.wait()
        densify(d, SUB)
        outs = fire_outs(p0, SUB)
    for ocp in outs:
        ocp.wait()

    @pl.when(j == NSUB - 1)
    def _tail():
        pltpu.sync_copy(xtail_hbm.at[b], tailb)
        flatten_tail(NVT)
        densify(TAIL_D, TAIL_NP)
        for ocp in fire_outs(TAIL_P0, TAIL_NP):
            ocp.wait()


def _tc_transpose_body(x_ref, o_ref):
    o_ref[0, 0] = jnp.transpose(x_ref[:, :PS], (1, 0))


_tc_transpose = pl.pallas_call(
    _tc_transpose_body,
    grid=(B, C),
    in_specs=[pl.BlockSpec((P, PSP), lambda b, c: (b * C + c, 0))],
    out_specs=pl.BlockSpec((1, 1, PS, P), lambda b, c: (b, c, 0, 0)),
    out_shape=jax.ShapeDtypeStruct((B, C, PS, P), jnp.float32),
)


def kernel(x):
    xtail = jnp.pad(x[:, :, TAIL_SRC:], ((0, 0), (0, 0), (0, TAIL_LEN - (T - TAIL_SRC))))
    patches2d, masks = _to_patches_sc(x, xtail)
    patches_t = _tc_transpose(patches2d)
    return patches_t.transpose(0, 1, 3, 2), masks


# speedup vs baseline: 72.1226x; 1.2767x over previous
import functools

import jax
import jax.numpy as jnp
from jax import lax
from jax.experimental import pallas as pl
from jax.experimental.pallas import tpu as pltpu
from jax.experimental.pallas import tpu_sc as plsc

B, C, T = 4, 8, 150050
PS, ST, P = 200, 150, 1000
NW = 32
LANES = 16

SUB = 16
NSUB = 8
WIN = SUB * NSUB
CLEN = 2688
NV = CLEN // LANES
CLAMP = 976
OUTC = SUB * PS

PSP = 208
TAIL_P0 = 992
TAIL_NP = P - TAIL_P0
TAIL_SRC = 148786
TAIL_D = TAIL_P0 * ST - TAIL_SRC
TAIL_LEN = 1280
NVT = TAIL_LEN // LANES

OFFS = [v * LANES for v in range(13)]

_mesh = plsc.VectorSubcoreMesh(core_axis_name="c", subcore_axis_name="s")


@functools.partial(
    pl.kernel,
    mesh=_mesh,
    out_type=[
        jax.ShapeDtypeStruct((B * C * P, PSP), jnp.float32),
        jax.ShapeDtypeStruct((B, C, P), jnp.float32),
    ],
    scratch_types=[
        pltpu.VMEM((2, C, CLEN), jnp.float32),
        pltpu.VMEM((C * CLEN,), jnp.float32),
        pltpu.VMEM((C, SUB, PSP), jnp.float32),
        pltpu.VMEM((C, P), jnp.float32),
        pltpu.VMEM((C, TAIL_LEN), jnp.float32),
        pltpu.SemaphoreType.DMA,
        pltpu.SemaphoreType.DMA,
    ],
)
def _to_patches_sc(
    x_hbm, xtail_hbm, patches_hbm, masks_hbm,
    in2, lin, outb, ones2d, tailb, sem_in, sem_out,
):
    wid = lax.axis_index("s") * 2 + lax.axis_index("c")
    b = wid // NSUB
    j = wid % NSUB

    ones = jnp.ones((LANES,), jnp.float32)

    @pl.when(j == 0)
    def _write_masks():
        def fill(i, carry):
            col = pl.multiple_of(i * LANES, LANES)
            for c in range(C):
                ones2d[c, pl.ds(col, LANES)] = ones
            return carry

        lax.fori_loop(0, P // LANES, fill, 0)
        for c in range(C):
            ones2d[c, pl.ds(P - LANES, LANES)] = ones
        pltpu.sync_copy(ones2d, masks_hbm.at[b])

    def flatten(bi, nvec):
        def fv(v, carry):
            col = pl.multiple_of(v * LANES, LANES)
            for c in range(C):
                lin[pl.ds(c * CLEN + col, LANES)] = in2[bi, c, pl.ds(col, LANES)]
            return carry

        lax.fori_loop(0, nvec, fv, 0)

    def flatten_tail(nvec):
        def fv(v, carry):
            col = pl.multiple_of(v * LANES, LANES)
            for c in range(C):
                lin[pl.ds(c * CLEN + col, LANES)] = tailb[c, pl.ds(col, LANES)]
            return carry

        lax.fori_loop(0, nvec, fv, 0)

    def densify(d, npatch):
        def fp(i, carry):
            src0 = d + i * ST
            for c in range(C):
                cb = c * CLEN + src0
                vals = [lin[pl.ds(cb + off, LANES)] for off in OFFS]
                for off, v in zip(OFFS, vals):
                    outb[c, i, pl.ds(off, LANES)] = v
            return carry

        lax.fori_loop(0, npatch, fp, 0)

    def fire_outs(p0, npatch):
        return [
            pltpu.async_copy(
                outb.at[c, pl.ds(0, npatch), :],
                patches_hbm.at[
                    pl.ds(pl.multiple_of((b * C + c) * P + p0, 8), npatch), :
                ],
                sem_out,
            )
            for c in range(C)
        ]

    def fire_load(s):
        p0 = jnp.minimum(WIN * j + SUB * s, CLAMP)
        src = p0 * ST
        cstart = pl.multiple_of((src // 128) * 128, 128)
        cp = pltpu.async_copy(
            x_hbm.at[b, :, pl.ds(cstart, CLEN)], in2.at[s % 2], sem_in
        )
        return cp, src - cstart, p0

    load = fire_load(0)
    outs = None
    for s in range(NSUB):
        cp, d, p0 = load
        cp.wait()
        if s + 1 < NSUB:
            load = fire_load(s + 1)
        flatten(s % 2, NV)
        if outs is not None:
            for ocp in outs:
                ocp.wait()
        densify(d, SUB)
        outs = fire_outs(p0, SUB)
    for ocp in outs:
        ocp.wait()

    @pl.when(j == NSUB - 1)
    def _tail():
        pltpu.sync_copy(xtail_hbm.at[b], tailb)
        flatten_tail(NVT)
        densify(TAIL_D, TAIL_NP)
        for ocp in fire_outs(TAIL_P0, TAIL_NP):
            ocp.wait()


def _tc_transpose_body(x_ref, o_ref):
    o_ref[0, 0] = jnp.transpose(x_ref[:, :PS], (1, 0))


_tc_transpose = pl.pallas_call(
    _tc_transpose_body,
    grid=(B, C),
    in_specs=[pl.BlockSpec((P, PSP), lambda b, c: (b * C + c, 0))],
    out_specs=pl.BlockSpec((1, 1, PS, P), lambda b, c: (b, c, 0, 0)),
    out_shape=jax.ShapeDtypeStruct((B, C, PS, P), jnp.float32),
)


def kernel(x):
    xtail = jnp.pad(x[:, :, TAIL_SRC:], ((0, 0), (0, 0), (0, TAIL_LEN - (T - TAIL_SRC))))
    patches2d, masks = _to_patches_sc(x, xtail)
    patches_t = _tc_transpose(patches2d)
    return patches_t.transpose(0, 1, 3, 2), masks


# speedup vs baseline: 92.9497x; 1.2888x over previous
import functools

import jax
import jax.numpy as jnp
from jax import lax
from jax.experimental import pallas as pl
from jax.experimental.pallas import tpu as pltpu
from jax.experimental.pallas import tpu_sc as plsc

B, C, T = 4, 8, 150050
PS, ST, P = 200, 150, 1000
NW = 32
LANES = 16

SUB = 16
NSUB = 8
WIN = SUB * NSUB
CLEN = 2688
NV = CLEN // LANES
CLAMP = 976
OUTC = SUB * PS

PSP = 208
TAIL_P0 = 992
TAIL_NP = P - TAIL_P0
TAIL_SRC = 148786
TAIL_D = TAIL_P0 * ST - TAIL_SRC
TAIL_LEN = 1280
NVT = TAIL_LEN // LANES

OFFS = [v * LANES for v in range(13)]

_mesh = plsc.VectorSubcoreMesh(core_axis_name="c", subcore_axis_name="s")


@functools.partial(
    pl.kernel,
    mesh=_mesh,
    out_type=[
        jax.ShapeDtypeStruct((B * C * P, PSP), jnp.float32),
        jax.ShapeDtypeStruct((B, C, P), jnp.float32),
    ],
    scratch_types=[
        pltpu.VMEM((2, C, CLEN), jnp.float32),
        pltpu.VMEM((C * CLEN,), jnp.float32),
        pltpu.VMEM((C, SUB, PSP), jnp.float32),
        pltpu.VMEM((C, P), jnp.float32),
        pltpu.VMEM((C, TAIL_LEN), jnp.float32),
        pltpu.SemaphoreType.DMA,
        pltpu.SemaphoreType.DMA,
    ],
)
def _to_patches_sc(
    x_hbm, xtail_hbm, patches_hbm, masks_hbm,
    in2, lin, outb, ones2d, tailb, sem_in, sem_out,
):
    wid = lax.axis_index("s") * 2 + lax.axis_index("c")
    b = wid // NSUB
    j = wid % NSUB

    ones = jnp.ones((LANES,), jnp.float32)

    @pl.when(j == 0)
    def _write_masks():
        def fill(i, carry):
            col = pl.multiple_of(i * LANES, LANES)
            for c in range(C):
                ones2d[c, pl.ds(col, LANES)] = ones
            return carry

        lax.fori_loop(0, P // LANES, fill, 0)
        for c in range(C):
            ones2d[c, pl.ds(P - LANES, LANES)] = ones
        pltpu.sync_copy(ones2d, masks_hbm.at[b])

    def flatten(bi, nvec):
        def fv(v, carry):
            col = pl.multiple_of(v * LANES, LANES)
            vals = [in2[bi, c, pl.ds(col, LANES)] for c in range(C)]
            for c in range(C):
                lin[pl.ds(c * CLEN + col, LANES)] = vals[c]
            return carry

        lax.fori_loop(0, nvec, fv, 0)

    def flatten_tail(nvec):
        def fv(v, carry):
            col = pl.multiple_of(v * LANES, LANES)
            vals = [tailb[c, pl.ds(col, LANES)] for c in range(C)]
            for c in range(C):
                lin[pl.ds(c * CLEN + col, LANES)] = vals[c]
            return carry

        lax.fori_loop(0, nvec, fv, 0)

    def densify(d, npatch):
        def fp(i, carry):
            src0 = d + i * ST
            for c in range(C):
                cb = c * CLEN + src0
                vals = [lin[pl.ds(cb + off, LANES)] for off in OFFS]
                for off, v in zip(OFFS, vals):
                    outb[c, i, pl.ds(off, LANES)] = v
            return carry

        lax.fori_loop(0, npatch, fp, 0)

    def fire_outs(p0, npatch):
        return [
            pltpu.async_copy(
                outb.at[c, pl.ds(0, npatch), :],
                patches_hbm.at[
                    pl.ds(pl.multiple_of((b * C + c) * P + p0, 8), npatch), :
                ],
                sem_out,
            )
            for c in range(C)
        ]

    def fire_load(s):
        p0 = jnp.minimum(WIN * j + SUB * s, CLAMP)
        src = p0 * ST
        cstart = pl.multiple_of((src // 128) * 128, 128)
        cp = pltpu.async_copy(
            x_hbm.at[b, :, pl.ds(cstart, CLEN)], in2.at[s % 2], sem_in
        )
        return cp, src - cstart, p0

    load = fire_load(0)
    outs = None
    for s in range(NSUB):
        cp, d, p0 = load
        cp.wait()
        if s + 1 < NSUB:
            load = fire_load(s + 1)
        flatten(s % 2, NV)
        if outs is not None:
            for ocp in outs:
                ocp.wait()
        densify(d, SUB)
        outs = fire_outs(p0, SUB)
    for ocp in outs:
        ocp.wait()

    @pl.when(j == NSUB - 1)
    def _tail():
        pltpu.sync_copy(xtail_hbm.at[b], tailb)
        flatten_tail(NVT)
        densify(TAIL_D, TAIL_NP)
        for ocp in fire_outs(TAIL_P0, TAIL_NP):
            ocp.wait()


def _tc_transpose_body(x_ref, o_ref):
    o_ref[0, 0] = jnp.transpose(x_ref[:, :PS], (1, 0))


_tc_transpose = pl.pallas_call(
    _tc_transpose_body,
    grid=(B, C),
    in_specs=[pl.BlockSpec((P, PSP), lambda b, c: (b * C + c, 0))],
    out_specs=pl.BlockSpec((1, 1, PS, P), lambda b, c: (b, c, 0, 0)),
    out_shape=jax.ShapeDtypeStruct((B, C, PS, P), jnp.float32),
)


def kernel(x):
    xtail = jnp.pad(x[:, :, TAIL_SRC:], ((0, 0), (0, 0), (0, TAIL_LEN - (T - TAIL_SRC))))
    patches2d, masks = _to_patches_sc(x, xtail)
    patches_t = _tc_transpose(patches2d)
    return patches_t.transpose(0, 1, 3, 2), masks
